# Initial kernel scaffold; baseline (speedup 1.0000x reference)
import jax
import jax.numpy as jnp
from jax.experimental import pallas as pl


def kernel(edge_index1, batch1, edge_index2, W1, b1, W2, b2, fc1_W, fc1_b, ln1_g, ln1_b, fc2_W, fc2_b, ln2_g, ln2_b, fc3_W, fc3_b):
    raise NotImplementedError("write your pallas kernel here")


# speedup vs baseline: 73.7389x; 73.7389x over previous
import functools

import jax
import jax.numpy as jnp
from jax import lax
from jax.experimental import pallas as pl
from jax.experimental.pallas import tpu as pltpu
from jax.experimental.pallas import tpu_sc as plsc

f32 = jnp.float32
i32 = jnp.int32

_N1, _E1 = 10000, 320000
_N2, _E2 = 400, 6400
_G, _K = 20, 50
_NP1, _NP2 = 10240, 512
_NT = 16


def _rsqrt(x):
    i = plsc.bitcast(x, i32)
    i = jnp.int32(0x5F3759DF) - (i >> 1)
    y = plsc.bitcast(i, f32)
    xh = x * 0.5
    for _ in range(3):
        y = y * (1.5 - xh * y * y)
    return y


def _zero(ref, nv):
    z = jnp.zeros((16,), f32)

    def b(j, c):
        ref[pl.ds(j * 16, 16)] = z
        return c

    lax.fori_loop(0, nv, b, None)


def _k1_graph(tid, src_hbm, dst_hbm, t_out, src_v, dst_v, gat_v, acc_v, acc2_v,
              sl_d, sl_g, sl_tmp, sl_red, sh_a, sh_b, sh_full, *, E, NP):
    ECH = E // _NT
    EV = ECH // 16
    SL = NP // _NT
    SV = SL // 16
    NV = NP // 16
    ebase = tid * ECH
    sbase = tid * SL

    pltpu.sync_copy(src_hbm.at[pl.ds(ebase, ECH)], src_v.at[pl.ds(0, ECH)])
    pltpu.sync_copy(dst_hbm.at[pl.ds(ebase, ECH)], dst_v.at[pl.ds(0, ECH)])

    def _reduce(sh, out_ref):
        _zero(out_ref, SV)

        def rc(c, carry):
            pltpu.sync_copy(sh.at[c, pl.ds(sbase, SL)], sl_tmp.at[pl.ds(0, SL)])

            def rj(j, cc):
                out_ref[pl.ds(j * 16, 16)] = (out_ref[pl.ds(j * 16, 16)]
                                              + sl_tmp[pl.ds(j * 16, 16)])
                return cc

            lax.fori_loop(0, SV, rj, None)
            return carry

        lax.fori_loop(0, _NT, rc, None)

    _zero(acc_v, NV)
    _zero(acc2_v, NV)
    ones = jnp.full((16,), 1.0, f32)

    def ha(i, c):
        sv = src_v[pl.ds(i * 16, 16)]
        dv = dst_v[pl.ds(i * 16, 16)]
        plsc.addupdate_scatter(acc_v, [sv], ones)
        plsc.addupdate_scatter(acc2_v, [dv], ones)
        return c

    lax.fori_loop(0, EV, ha, None)
    pltpu.sync_copy(acc_v.at[pl.ds(0, NP)], sh_a.at[tid, pl.ds(0, NP)])
    pltpu.sync_copy(acc2_v.at[pl.ds(0, NP)], sh_b.at[tid, pl.ds(0, NP)])
    plsc.subcore_barrier()

    _reduce(sh_a, sl_red)
    _reduce(sh_b, sl_g)

    def pa(j, c):
        x = sl_red[pl.ds(j * 16, 16)]
        deg = sl_g[pl.ds(j * 16, 16)] + 1.0
        dinv = _rsqrt(deg)
        sl_d[pl.ds(j * 16, 16)] = dinv
        sl_g[pl.ds(j * 16, 16)] = x * dinv
        return c

    lax.fori_loop(0, SV, pa, None)
    pltpu.sync_copy(sl_g.at[pl.ds(0, SL)], sh_full.at[pl.ds(sbase, SL)])
    plsc.subcore_barrier()

    for phase in (0, 1):
        pltpu.sync_copy(sh_full.at[pl.ds(0, NP)], gat_v.at[pl.ds(0, NP)])
        _zero(acc_v, NV)

        def gs(i, c):
            sv = src_v[pl.ds(i * 16, 16)]
            dv = dst_v[pl.ds(i * 16, 16)]
            vals = plsc.load_gather(gat_v, [sv])
            plsc.addupdate_scatter(acc_v, [dv], vals)
            return c

        lax.fori_loop(0, EV, gs, None)
        pltpu.sync_copy(acc_v.at[pl.ds(0, NP)], sh_a.at[tid, pl.ds(0, NP)])
        plsc.subcore_barrier()
        _reduce(sh_a, sl_red)

        def pb(j, c):
            agg = sl_red[pl.ds(j * 16, 16)]
            dinv = sl_d[pl.ds(j * 16, 16)]
            gv = sl_g[pl.ds(j * 16, 16)]
            sval = dinv * (agg + gv)
            if phase == 0:
                sl_g[pl.ds(j * 16, 16)] = sval * dinv
            else:
                sl_g[pl.ds(j * 16, 16)] = sval
            return c

        lax.fori_loop(0, SV, pb, None)
        if phase == 0:
            pltpu.sync_copy(sl_g.at[pl.ds(0, SL)], sh_full.at[pl.ds(sbase, SL)])
            plsc.subcore_barrier()
        else:
            pltpu.sync_copy(sl_g.at[pl.ds(0, SL)], t_out.at[pl.ds(sbase, SL)])


def _k1_body(src1, dst1, src2, dst2, t1_out, t2_out, src_v, dst_v, gat_v,
             acc_v, acc2_v, sl_d, sl_g, sl_tmp, sl_red, sh_a, sh_b, sh_full):
    cid = lax.axis_index("c")
    tid = lax.axis_index("s")
    args = (src_v, dst_v, gat_v, acc_v, acc2_v, sl_d, sl_g, sl_tmp, sl_red,
            sh_a, sh_b, sh_full)

    @pl.when(cid == 0)
    def _():
        _k1_graph(tid, src1, dst1, t1_out, *args, E=_E1, NP=_NP1)

    @pl.when(cid == 1)
    def _():
        _k1_graph(tid, src2, dst2, t2_out, *args, E=_E2, NP=_NP2)


_k1 = functools.partial(
    pl.kernel,
    out_type=(jax.ShapeDtypeStruct((_NP1,), f32),
              jax.ShapeDtypeStruct((_NP2,), f32)),
    compiler_params=pltpu.CompilerParams(needs_layout_passes=False),
    mesh=plsc.VectorSubcoreMesh(core_axis_name="c", subcore_axis_name="s"),
    scratch_types=(
        pltpu.VMEM((_E1 // _NT,), i32),
        pltpu.VMEM((_E1 // _NT,), i32),
        pltpu.VMEM((_NP1,), f32),
        pltpu.VMEM((_NP1,), f32),
        pltpu.VMEM((_NP1,), f32),
        pltpu.VMEM((_NP1 // _NT,), f32),
        pltpu.VMEM((_NP1 // _NT,), f32),
        pltpu.VMEM((_NP1 // _NT,), f32),
        pltpu.VMEM((_NP1 // _NT,), f32),
        pltpu.VMEM_SHARED((_NT, _NP1), f32),
        pltpu.VMEM_SHARED((_NT, _NP1), f32),
        pltpu.VMEM_SHARED((_NP1,), f32),
    ),
)(_k1_body)


def _k2_body(t1p, t2p, batch, sel_out, val_out,
             t1_v, batch_v, keys_v, c16, selbuf, vbuf):
    cid = lax.axis_index("c")
    tid = lax.axis_index("s")
    g = cid * _NT + tid

    @pl.when(g < _G)
    def _():
        pltpu.sync_copy(batch, batch_v.at[pl.ds(0, _N1)])
        pltpu.sync_copy(t1p, t1_v)
        pltpu.sync_copy(t2p.at[pl.ds(_N2 - 16, 16)], c16)
        iota = lax.iota(i32, 16)
        cval = jnp.sum(jnp.where(iota == 15, c16[pl.ds(0, 16)], 0.0))

        def bb(i, carry):
            lt, le = carry
            bv = batch_v[pl.ds(i * 16, 16)]
            lt = lt + jnp.where(bv < g, 1, 0).astype(i32)
            le = le + jnp.where(bv <= g, 1, 0).astype(i32)
            return lt, le

        z16 = jnp.zeros((16,), i32)
        lt, le = lax.fori_loop(0, _N1 // 16, bb, (z16, z16))
        start = jnp.sum(lt)
        end = jnp.sum(le)
        js = start >> 4
        je = (end + 15) >> 4

        def kb(j, c):
            gi = j * 16 + iota
            t1v = t1_v[pl.ds(j * 16, 16)]
            d = t1v - cval
            k = d * d
            inside = (gi >= start) & (gi < end)
            keys_v[pl.ds(j * 16, 16)] = jnp.where(inside, k, -1.0)
            return c

        lax.fori_loop(js, je, kb, None)
        _zero(selbuf, 4)
        _zero(vbuf, 4)
        big = jnp.int32(1 << 30)

        def ext(r, c):
            def mx(j, m):
                return jnp.maximum(m, keys_v[pl.ds(j * 16, 16)])

            mv = lax.fori_loop(js, je, mx, jnp.full((16,), -2.0, f32))
            m = jnp.max(mv)

            def fi(j, acc):
                kv = keys_v[pl.ds(j * 16, 16)]
                gi = j * 16 + iota
                return jnp.minimum(acc, jnp.where(kv == m, gi, big))

            iv = lax.fori_loop(js, je, fi, jnp.full((16,), big, i32))
            idx = jnp.min(iv)
            validf = jnp.where(m >= 0.0, 1.0, 0.0).astype(f32)
            idx = jnp.minimum(idx, jnp.int32(_NP1 - 1))
            av = idx & jnp.int32(-16)
            lane = idx & jnp.int32(15)
            t1vv = t1_v[pl.ds(av, 16)]
            val = jnp.sum(jnp.where(iota == lane, t1vv, 0.0))
            kvv = keys_v[pl.ds(av, 16)]
            keys_v[pl.ds(av, 16)] = jnp.where(iota == lane, -1.0, kvv)
            rb = r & jnp.int32(-16)
            rl = r & jnp.int32(15)
            sb = selbuf[pl.ds(rb, 16)]
            selbuf[pl.ds(rb, 16)] = jnp.where(iota == rl, val * validf, sb)
            vb = vbuf[pl.ds(rb, 16)]
            vbuf[pl.ds(rb, 16)] = jnp.where(iota == rl, validf, vb)
            return c

        lax.fori_loop(0, _K, ext, None)
        pltpu.sync_copy(selbuf, sel_out.at[g])
        pltpu.sync_copy(vbuf, val_out.at[g])


_k2 = functools.partial(
    pl.kernel,
    out_type=(jax.ShapeDtypeStruct((_G, 64), f32),
              jax.ShapeDtypeStruct((_G, 64), f32)),
    compiler_params=pltpu.CompilerParams(needs_layout_passes=False),
    mesh=plsc.VectorSubcoreMesh(core_axis_name="c", subcore_axis_name="s"),
    scratch_types=(
        pltpu.VMEM((_NP1,), f32),
        pltpu.VMEM((_N1,), i32),
        pltpu.VMEM((_NP1,), f32),
        pltpu.VMEM((16,), f32),
        pltpu.VMEM((64,), f32),
        pltpu.VMEM((64,), f32),
    ),
)(_k2_body)


def _k3_body(t1s_ref, vm_ref, t2_ref, w1_ref, w2_ref, fc1w_ref, fc1b_ref,
             g1_ref, b1_ref, fc2w_ref, fc2b_ref, g2_ref, b2_ref,
             fc3w_ref, fc3b_ref, o_ref):
    w1r = jnp.maximum(w1_ref[...], 0.0)
    u = jnp.maximum(jnp.dot(w1r, w2_ref[...],
                            preferred_element_type=f32), 0.0)
    nu2 = jnp.sum(u * u)
    t2 = t2_ref[...]
    t1s = t1s_ref[...]
    vm = vm_ref[...]

    h = jnp.zeros((_G, 128), f32)
    for r in range(_K):
        tcol = t1s[:, r:r + 1]
        vcol = vm[:, r:r + 1]
        d2 = (tcol - t2) ** 2 * nu2
        dr = vcol * jnp.sqrt(jnp.maximum(d2, 1e-12))
        h = h + jnp.dot(dr, fc1w_ref[r * _N2:(r + 1) * _N2, :],
                        preferred_element_type=f32)
    h = h + fc1b_ref[...]

    def ln(x, gg, bb):
        mu = jnp.mean(x, axis=1, keepdims=True)
        va = jnp.mean((x - mu) ** 2, axis=1, keepdims=True)
        return (x - mu) / jnp.sqrt(va + 1e-5) * gg + bb

    h = jnp.maximum(ln(h, g1_ref[...], b1_ref[...]), 0.0)
    h = jnp.dot(h, fc2w_ref[...], preferred_element_type=f32) + fc2b_ref[...]
    h = jnp.maximum(ln(h, g2_ref[...], b2_ref[...]), 0.0)
    o = jnp.dot(h, fc3w_ref[...], preferred_element_type=f32) + fc3b_ref[...]
    o_ref[...] = 1.0 / (1.0 + jnp.exp(-o))


_k3 = pl.pallas_call(
    _k3_body,
    out_shape=jax.ShapeDtypeStruct((_G, 1), f32),
)


def kernel(edge_index1, batch1, edge_index2, W1, b1, W2, b2,
           fc1_W, fc1_b, ln1_g, ln1_b, fc2_W, fc2_b, ln2_g, ln2_b,
           fc3_W, fc3_b):
    src1 = edge_index1[0].astype(i32)
    dst1 = edge_index1[1].astype(i32)
    src2 = edge_index2[0].astype(i32)
    dst2 = edge_index2[1].astype(i32)
    batch = batch1.astype(i32)

    t1p, t2p = _k1(src1, dst1, src2, dst2)
    t1sel, validm = _k2(t1p, t2p, batch)
    out = _k3(t1sel, validm, t2p[:_N2].reshape(1, _N2),
              W1.reshape(1, 128), W2,
              fc1_W, fc1_b.reshape(1, 128), ln1_g.reshape(1, 128),
              ln1_b.reshape(1, 128), fc2_W, fc2_b.reshape(1, 64),
              ln2_g.reshape(1, 64), ln2_b.reshape(1, 64), fc3_W,
              fc3_b.reshape(1, 1))
    return out


# speedup vs baseline: 120.2083x; 1.6302x over previous
import functools

import jax
import jax.numpy as jnp
from jax import lax
from jax.experimental import pallas as pl
from jax.experimental.pallas import tpu as pltpu
from jax.experimental.pallas import tpu_sc as plsc

f32 = jnp.float32
i32 = jnp.int32

_N1, _E1 = 10000, 320000
_N2, _E2 = 400, 6400
_G, _K = 20, 50
_NP1, _NP2 = 10240, 512
_NT = 16


def _rsqrt(x):
    i = plsc.bitcast(x, i32)
    i = jnp.int32(0x5F3759DF) - (i >> 1)
    y = plsc.bitcast(i, f32)
    xh = x * 0.5
    for _ in range(3):
        y = y * (1.5 - xh * y * y)
    return y


def _zero(ref, nv):
    z = jnp.zeros((16,), f32)

    def b(j, c):
        ref[pl.ds(j * 16, 16)] = z
        return c

    lax.fori_loop(0, nv, b, None)


def _graph_pipeline(tid, edge_hbm, sh_t, src_v, dst_v, gat_v, acc_v, acc2_v,
                    sl_d, sl_g, red2_v, sh_a, sh_b, sh_full, *, E, NP):
    ECH = E // _NT
    EV = ECH // 16
    SL = NP // _NT
    SV = SL // 16
    NV = NP // 16
    ebase = tid * ECH
    sbase = tid * SL

    pltpu.sync_copy(edge_hbm.at[pl.ds(ebase, ECH)], src_v.at[pl.ds(0, ECH)])
    pltpu.sync_copy(edge_hbm.at[pl.ds(E + ebase, ECH)], dst_v.at[pl.ds(0, ECH)])

    def _reduce(sh, red2_v, out_ref):
        if SL % 128 == 0:
            pltpu.sync_copy(sh.at[:, pl.ds(sbase, SL)], red2_v.at[:, pl.ds(0, SL)])
            rbase = 0
        else:
            pltpu.sync_copy(sh.at[:, pl.ds(0, NP)], red2_v.at[:, pl.ds(0, NP)])
            rbase = sbase

        def rj(j, cc):
            acc = red2_v[0, pl.ds(rbase + j * 16, 16)]
            for c in range(1, _NT):
                acc = acc + red2_v[c, pl.ds(rbase + j * 16, 16)]
            out_ref[pl.ds(j * 16, 16)] = acc
            return cc

        lax.fori_loop(0, SV, rj, None)

    _zero(acc_v, NV)
    _zero(acc2_v, NV)
    ones = jnp.full((16,), 1.0, f32)

    @plsc.parallel_loop(0, ECH, step=16, unroll=4)
    def _(e):
        sv = src_v[pl.ds(e, 16)]
        dv = dst_v[pl.ds(e, 16)]
        plsc.addupdate_scatter(acc_v, [sv], ones)
        plsc.addupdate_scatter(acc2_v, [dv], ones)
    pltpu.sync_copy(acc_v.at[pl.ds(0, NP)], sh_a.at[tid, pl.ds(0, NP)])
    pltpu.sync_copy(acc2_v.at[pl.ds(0, NP)], sh_b.at[tid, pl.ds(0, NP)])
    plsc.subcore_barrier()

    _reduce(sh_a, red2_v, gat_v)
    _reduce(sh_b, red2_v, sl_g)

    def pa(j, c):
        x = gat_v[pl.ds(j * 16, 16)]
        deg = sl_g[pl.ds(j * 16, 16)] + 1.0
        dinv = _rsqrt(deg)
        sl_d[pl.ds(j * 16, 16)] = dinv
        sl_g[pl.ds(j * 16, 16)] = x * dinv
        return c

    lax.fori_loop(0, SV, pa, None)
    pltpu.sync_copy(sl_g.at[pl.ds(0, SL)], sh_full.at[pl.ds(sbase, SL)])
    plsc.subcore_barrier()

    for phase in (0, 1):
        pltpu.sync_copy(sh_full.at[pl.ds(0, NP)], gat_v.at[pl.ds(0, NP)])
        _zero(acc_v, NV)

        @plsc.parallel_loop(0, ECH, step=16, unroll=4)
        def _(e):
            sv = src_v[pl.ds(e, 16)]
            dv = dst_v[pl.ds(e, 16)]
            vals = plsc.load_gather(gat_v, [sv])
            plsc.addupdate_scatter(acc_v, [dv], vals)
        pltpu.sync_copy(acc_v.at[pl.ds(0, NP)], sh_a.at[tid, pl.ds(0, NP)])
        plsc.subcore_barrier()
        _reduce(sh_a, red2_v, acc2_v)

        def pb(j, c):
            agg = acc2_v[pl.ds(j * 16, 16)]
            dinv = sl_d[pl.ds(j * 16, 16)]
            gv = sl_g[pl.ds(j * 16, 16)]
            sval = dinv * (agg + gv)
            if phase == 0:
                sl_g[pl.ds(j * 16, 16)] = sval * dinv
            else:
                sl_g[pl.ds(j * 16, 16)] = sval
            return c

        lax.fori_loop(0, SV, pb, None)
        if phase == 0:
            pltpu.sync_copy(sl_g.at[pl.ds(0, SL)], sh_full.at[pl.ds(sbase, SL)])
        else:
            pltpu.sync_copy(sl_g.at[pl.ds(0, SL)], sh_t.at[pl.ds(sbase, SL)])
        plsc.subcore_barrier()


def _km_body(edges1, edges2, batch, sel_out, val_out, t2_out,
             src_v, dst_v, gat_v, acc_v, acc2_v, sl_d, sl_g, red2_v,
             batch_v, selbuf, vbuf, c16, sh_a, sh_b, sh_full, sh_t1, sh_t2):
    cid = lax.axis_index("c")
    tid = lax.axis_index("s")
    args = (src_v, dst_v, gat_v, acc_v, acc2_v, sl_d, sl_g, red2_v,
            sh_a, sh_b, sh_full)

    _graph_pipeline(tid, edges1, sh_t1, *args, E=_E1, NP=_NP1)
    _graph_pipeline(tid, edges2, sh_t2, *args, E=_E2, NP=_NP2)

    @pl.when(cid == 0)
    def _():
        sl2 = _NP2 // _NT
        pltpu.sync_copy(sl_g.at[pl.ds(0, sl2)],
                        t2_out.at[pl.ds(tid * sl2, sl2)])

    g = cid * _NT + tid

    @pl.when(g < _G)
    def _():
        pltpu.sync_copy(batch, batch_v.at[pl.ds(0, _N1)])
        pltpu.sync_copy(sh_t1.at[pl.ds(0, _NP1)], gat_v.at[pl.ds(0, _NP1)])
        pltpu.sync_copy(sh_t2.at[pl.ds(_N2 - 16, 16)], c16)
        iota = lax.iota(i32, 16)
        cval = jnp.sum(jnp.where(iota == 15, c16[pl.ds(0, 16)], 0.0))

        def bb(i, carry):
            lt, le = carry
            bv = batch_v[pl.ds(i * 16, 16)]
            lt = lt + jnp.where(bv < g, 1, 0).astype(i32)
            le = le + jnp.where(bv <= g, 1, 0).astype(i32)
            return lt, le

        z16 = jnp.zeros((16,), i32)
        lt, le = lax.fori_loop(0, _N1 // 16, bb, (z16, z16), unroll=4)
        start = jnp.sum(lt)
        end = jnp.sum(le)
        js = start >> 4
        je = (end + 15) >> 4

        def kb(j, c):
            gi = j * 16 + iota
            t1v = gat_v[pl.ds(j * 16, 16)]
            d = t1v - cval
            k = d * d
            inside = (gi >= start) & (gi < end)
            acc_v[pl.ds(j * 16, 16)] = jnp.where(inside, k, -1.0)
            return c

        lax.fori_loop(js, je, kb, None)
        _zero(selbuf, 4)
        _zero(vbuf, 4)
        big = jnp.int32(1 << 30)

        def ext(r, c):
            def mx(j, carry):
                mv, miv = carry
                kv = acc_v[pl.ds(j * 16, 16)]
                gi = j * 16 + iota
                better = kv > mv
                return (jnp.where(better, kv, mv),
                        jnp.where(better, gi, miv))

            mv, miv = lax.fori_loop(
                js, je, mx,
                (jnp.full((16,), -2.0, f32), jnp.full((16,), big, i32)))
            m = jnp.max(mv)
            idx = jnp.min(jnp.where(mv == m, miv, big))
            validf = jnp.where(m >= 0.0, 1.0, 0.0).astype(f32)
            idx = jnp.minimum(idx, jnp.int32(_NP1 - 1))
            av = idx & jnp.int32(-16)
            lane = idx & jnp.int32(15)
            t1vv = gat_v[pl.ds(av, 16)]
            val = jnp.sum(jnp.where(iota == lane, t1vv, 0.0))
            kvv = acc_v[pl.ds(av, 16)]
            acc_v[pl.ds(av, 16)] = jnp.where(iota == lane, -1.0, kvv)
            rb = r & jnp.int32(-16)
            rl = r & jnp.int32(15)
            sb = selbuf[pl.ds(rb, 16)]
            selbuf[pl.ds(rb, 16)] = jnp.where(iota == rl, val * validf, sb)
            vb = vbuf[pl.ds(rb, 16)]
            vbuf[pl.ds(rb, 16)] = jnp.where(iota == rl, validf, vb)
            return c

        lax.fori_loop(0, _K, ext, None)
        pltpu.sync_copy(selbuf, sel_out.at[g])
        pltpu.sync_copy(vbuf, val_out.at[g])


_km = functools.partial(
    pl.kernel,
    out_type=(jax.ShapeDtypeStruct((_G, 64), f32),
              jax.ShapeDtypeStruct((_G, 64), f32),
              jax.ShapeDtypeStruct((_NP2,), f32)),
    compiler_params=pltpu.CompilerParams(needs_layout_passes=False),
    mesh=plsc.VectorSubcoreMesh(core_axis_name="c", subcore_axis_name="s"),
    scratch_types=(
        pltpu.VMEM((_E1 // _NT,), i32),
        pltpu.VMEM((_E1 // _NT,), i32),
        pltpu.VMEM((_NP1,), f32),
        pltpu.VMEM((_NP1,), f32),
        pltpu.VMEM((_NP1,), f32),
        pltpu.VMEM((_NP1 // _NT,), f32),
        pltpu.VMEM((_NP1 // _NT,), f32),
        pltpu.VMEM((_NT, _NP1 // _NT), f32),
        pltpu.VMEM((_N1,), i32),
        pltpu.VMEM((64,), f32),
        pltpu.VMEM((64,), f32),
        pltpu.VMEM((16,), f32),
        pltpu.VMEM_SHARED((_NT, _NP1), f32),
        pltpu.VMEM_SHARED((_NT, _NP1), f32),
        pltpu.VMEM_SHARED((_NP1,), f32),
        pltpu.VMEM_SHARED((_NP1,), f32),
        pltpu.VMEM_SHARED((_NP2,), f32),
    ),
)(_km_body)


def _k3_body(t1s_ref, vm_ref, t2_ref, w1_ref, w2_ref, fc1w_ref, fc1b_ref,
             g1_ref, b1_ref, fc2w_ref, fc2b_ref, g2_ref, b2_ref,
             fc3w_ref, fc3b_ref, o_ref):
    w1r = jnp.maximum(w1_ref[...], 0.0)
    u = jnp.maximum(jnp.dot(w1r, w2_ref[...],
                            preferred_element_type=f32), 0.0)
    nu2 = jnp.sum(u * u)
    t2 = t2_ref[...][:, :_N2]
    t1s = t1s_ref[...]
    vm = vm_ref[...]

    h = jnp.zeros((_G, 128), f32)
    for r in range(_K):
        tcol = t1s[:, r:r + 1]
        vcol = vm[:, r:r + 1]
        d2 = (tcol - t2) ** 2 * nu2
        dr = vcol * jnp.sqrt(jnp.maximum(d2, 1e-12))
        h = h + jnp.dot(dr, fc1w_ref[r * _N2:(r + 1) * _N2, :],
                        preferred_element_type=f32)
    h = h + fc1b_ref[...]

    def ln(x, gg, bb):
        mu = jnp.mean(x, axis=1, keepdims=True)
        va = jnp.mean((x - mu) ** 2, axis=1, keepdims=True)
        return (x - mu) / jnp.sqrt(va + 1e-5) * gg + bb

    h = jnp.maximum(ln(h, g1_ref[...], b1_ref[...]), 0.0)
    h = jnp.dot(h, fc2w_ref[...], preferred_element_type=f32) + fc2b_ref[...]
    h = jnp.maximum(ln(h, g2_ref[...], b2_ref[...]), 0.0)
    o = jnp.dot(h, fc3w_ref[...], preferred_element_type=f32) + fc3b_ref[...]
    o_ref[...] = 1.0 / (1.0 + jnp.exp(-o))


_k3 = pl.pallas_call(
    _k3_body,
    out_shape=jax.ShapeDtypeStruct((_G, 1), f32),
)


def kernel(edge_index1, batch1, edge_index2, W1, b1, W2, b2,
           fc1_W, fc1_b, ln1_g, ln1_b, fc2_W, fc2_b, ln2_g, ln2_b,
           fc3_W, fc3_b):
    t1sel, validm, t2p = _km(edge_index1.astype(i32).reshape(-1),
                             edge_index2.astype(i32).reshape(-1),
                             batch1.astype(i32))
    out = _k3(t1sel, validm, t2p.reshape(1, _NP2),
              W1.reshape(1, 128), W2,
              fc1_W, fc1_b.reshape(1, 128), ln1_g.reshape(1, 128),
              ln1_b.reshape(1, 128), fc2_W, fc2_b.reshape(1, 64),
              ln2_g.reshape(1, 64), ln2_b.reshape(1, 64), fc3_W,
              fc3_b.reshape(1, 1))
    return out
